# Initial kernel scaffold; baseline (speedup 1.0000x reference)
import jax
import jax.numpy as jnp
from jax.experimental import pallas as pl


def kernel(src, dst):
    raise NotImplementedError("write your pallas kernel here")


# speedup vs baseline: 3.0026x; 3.0026x over previous
import functools

import jax
import jax.numpy as jnp
from jax.experimental import pallas as pl
from jax.experimental.pallas import tpu as pltpu

K_NN = 16
I32MAX = 2**31 - 1


def _knn_body(src_ref, dst_ref, vals_ref, idx_ref, *, n_total, k):
    src = src_ref[...]
    dst = dst_ref[...]
    qb = src.shape[0]
    s2 = jnp.sum(src * src, axis=-1, keepdims=True)
    d2 = jnp.sum(dst * dst, axis=-1)[None, :]
    ab = jax.lax.dot_general(src, dst, (((1,), (1,)), ((), ())),
                             preferred_element_type=jnp.float32)
    dist2 = jnp.maximum(s2 + d2 - 2.0 * ab, 0.0)
    safe = jnp.where(dist2 > 0, dist2, 1.0)
    vals = jnp.where(dist2 > 0, jnp.sqrt(safe), 0.0)
    cols = jax.lax.broadcasted_iota(jnp.int32, (qb, n_total), 1)
    out_v, out_i = [], []
    for t in range(k):
        m = jnp.min(vals, axis=1, keepdims=True)
        im = jnp.min(jnp.where(vals == m, cols, I32MAX), axis=1,
                     keepdims=True)
        out_v.append(m)
        out_i.append(im)
        if t < k - 1:
            vals = jnp.where(cols == im, jnp.inf, vals)
    vals_ref[...] = jnp.concatenate(out_v, axis=1)
    idx_ref[...] = jnp.concatenate(out_i, axis=1)


@functools.partial(jax.jit, static_argnames=())
def kernel(src, dst):
    q, d = src.shape
    n, _ = dst.shape
    qb = min(128, q)
    body = functools.partial(_knn_body, n_total=n, k=K_NN)
    vals, idx = pl.pallas_call(
        body,
        grid=(q // qb,),
        in_specs=[
            pl.BlockSpec((qb, d), lambda i: (i, 0)),
            pl.BlockSpec((n, d), lambda i: (0, 0)),
        ],
        out_specs=[
            pl.BlockSpec((qb, K_NN), lambda i: (i, 0)),
            pl.BlockSpec((qb, K_NN), lambda i: (i, 0)),
        ],
        out_shape=[
            jax.ShapeDtypeStruct((q, K_NN), jnp.float32),
            jax.ShapeDtypeStruct((q, K_NN), jnp.int32),
        ],
        compiler_params=pltpu.CompilerParams(
            dimension_semantics=("arbitrary",),
        ),
    )(src, dst)
    return vals, idx
